# baseline (speedup 1.0000x reference)
import functools

import jax
import jax.numpy as jnp
from jax import lax
from jax.experimental import pallas as pl
from jax.experimental.pallas import tpu as pltpu
from jax.experimental.pallas import tpu_sc as plsc

_NC = 2
_NS = 16
_NW = _NC * _NS


def _argmin_body(z_ref, cb_ref, idx_ref, cb128_ref):
    zb = z_ref[...]
    cb = cb_ref[...]
    zbt = zb.T
    zn = jnp.sum(zbt * zbt, axis=0, keepdims=True)
    cn = jnp.sum(cb * cb, axis=1, keepdims=True)
    d2 = cn - 2.0 * jnp.dot(cb, zbt, preferred_element_type=jnp.float32) + zn
    d2 = jnp.maximum(d2, 0.0)
    m = jnp.min(d2, axis=0, keepdims=True)
    iota = lax.broadcasted_iota(jnp.int32, d2.shape, 0)
    idx_ref[...] = jnp.min(jnp.where(d2 == m, iota, jnp.int32(1 << 30)), axis=0)

    @pl.when(pl.program_id(0) == 0)
    def _():
        cb128_ref[...] = jnp.concatenate(
            [cb, jnp.zeros((cb.shape[0], 128 - cb.shape[1]), jnp.float32)],
            axis=1)


def _argmin_tc(z2d, cb, block=256, interpret=False):
    n = z2d.shape[0]
    k = cb.shape[0]
    return pl.pallas_call(
        _argmin_body,
        grid=(n // block,),
        in_specs=[
            pl.BlockSpec((block, z2d.shape[1]), lambda i: (i, 0)),
            pl.BlockSpec((k, cb.shape[1]), lambda i: (0, 0)),
        ],
        out_specs=[
            pl.BlockSpec((block,), lambda i: (i,)),
            pl.BlockSpec((k, 128), lambda i: (0, 0)),
        ],
        out_shape=[
            jax.ShapeDtypeStruct((n,), jnp.int32),
            jax.ShapeDtypeStruct((k, 128), jnp.float32),
        ],
        interpret=interpret,
    )(z2d, cb)


def _gather_sc(table128, idx):
    b = idx.shape[0]
    bpw = b // _NW
    mesh = plsc.VectorSubcoreMesh(core_axis_name="c", subcore_axis_name="s")

    @functools.partial(
        pl.kernel,
        mesh=mesh,
        out_type=jax.ShapeDtypeStruct((b, 128), jnp.float32),
        scratch_types=[
            pltpu.VMEM((bpw,), jnp.int32),
            pltpu.VMEM((bpw, 128), jnp.float32),
            pltpu.SemaphoreType.DMA,
        ],
    )
    def gk(table_hbm, idx_hbm, out_hbm, idx_v, rows_v, sem):
        wid = lax.axis_index("s") * _NC + lax.axis_index("c")
        base = wid * bpw
        pltpu.sync_copy(idx_hbm.at[pl.ds(base, bpw)], idx_v)
        pltpu.async_copy(table_hbm.at[idx_v], rows_v, sem).wait()
        pltpu.sync_copy(rows_v, out_hbm.at[pl.ds(base, bpw)])

    return gk(table128, idx)


def kernel(z, codebook):
    d = codebook.shape[1]
    z2d = z.reshape(-1, d)
    idx, cb128 = _argmin_tc(z2d, codebook)
    out128 = _gather_sc(cb128, idx)
    return out128[:, :d].reshape(z.shape)


# speedup vs baseline: 2.1734x; 2.1734x over previous
import functools

import jax
import jax.numpy as jnp
from jax import lax
from jax.experimental import pallas as pl
from jax.experimental.pallas import tpu as pltpu
from jax.experimental.pallas import tpu_sc as plsc

_NC = 2
_NS = 16
_NW = _NC * _NS


def _vq_body(z_ref, cbt_ref, cb_ref, out_ref):
    zb = z_ref[...]
    cbt = cbt_ref[...]
    cb = cb_ref[...]
    zn = jnp.sum(zb * zb, axis=1, keepdims=True)
    cn = jnp.sum(cbt * cbt, axis=0, keepdims=True)
    d2 = zn - 2.0 * jnp.dot(zb, cbt, preferred_element_type=jnp.float32) + cn
    d2 = jnp.maximum(d2, 0.0)
    m = jnp.min(d2, axis=1, keepdims=True)
    iota = lax.broadcasted_iota(jnp.int32, d2.shape, 1)
    idx = jnp.min(jnp.where(d2 == m, iota, jnp.int32(1 << 30)),
                  axis=1, keepdims=True)
    onehot = (iota == idx).astype(jnp.float32)
    out_ref[...] = jnp.dot(onehot, cb, preferred_element_type=jnp.float32)


def _vq_tc(z2d, cb, block=256, interpret=False):
    n, d = z2d.shape
    k = cb.shape[0]
    return pl.pallas_call(
        _vq_body,
        grid=(n // block,),
        in_specs=[
            pl.BlockSpec((block, d), lambda i: (i, 0)),
            pl.BlockSpec((d, k), lambda i: (0, 0)),
            pl.BlockSpec((k, d), lambda i: (0, 0)),
        ],
        out_specs=pl.BlockSpec((block, d), lambda i: (i, 0)),
        out_shape=jax.ShapeDtypeStruct((n, d), jnp.float32),
        interpret=interpret,
    )(z2d, cb.T, cb)


def kernel(z, codebook):
    d = codebook.shape[1]
    z2d = z.reshape(-1, d)
    return _vq_tc(z2d, codebook).reshape(z.shape)
